# baseline (speedup 1.0000x reference)
import jax
import jax.numpy as jnp
from jax import lax
from jax.experimental import pallas as pl
from jax.experimental.pallas import tpu as pltpu
from jax.experimental.pallas import tpu_sc as plsc

N_NODES = 100000
E = 3200000
G = 16384
MAX_STEPS = 10

NC = 2
NS = 16
NW = NC * NS
L = 16

MCHUNK = 5120
NCHUNKS = E // MCHUNK
QG = 2048
NQ = G // QG

def _fori(n, body):
    lax.fori_loop(jnp.asarray(0, jnp.int32), jnp.asarray(n, jnp.int32),
                  body, None)


def _body(eif, eb, curr, stepc, done32, act,
          mask_o, curr_o, step_o, done_o,
          packed, hs, es, ms, tbl, idxq, idxt, stp, dne, tmp16, sem,
          s_h, s_e, s_o0, s_o1):
    ii = lax.iota(jnp.int32, L)
    zc = jnp.zeros((L,), jnp.int32)
    c0 = jnp.asarray(0, jnp.int32)
    c1 = jnp.asarray(1, jnp.int32)
    wid = (jnp.asarray(lax.axis_index("s"), jnp.int32) * NC
           + jnp.asarray(lax.axis_index("c"), jnp.int32))

    if True:
        for pc in range(8):
            base = pc * 2048
            pltpu.sync_copy(curr.at[pl.ds(base, 2048)], hs.at[pl.ds(0, 2048)])
            pltpu.sync_copy(stepc.at[pl.ds(base, 2048)],
                            hs.at[pl.ds(2048, 2048)])
            pltpu.sync_copy(done32.at[pl.ds(base, 2048)],
                            hs.at[pl.ds(4096, 2048)])

            def pbody(v, carry, base=base):
                o = 16 * v
                c = hs[pl.ds(o, 16)]
                s = hs[pl.ds(2048 + o, 16)]
                d = hs[pl.ds(4096 + o, 16)]
                active = (d == 0) & (s < MAX_STEPS)
                packed[pl.ds(base + o, 16)] = jnp.where(active, c, -1)
                return carry

            _fori(2048 // 16, pbody)

    if True:
        go = wid * 512
        pltpu.sync_copy(act.at[pl.ds(go, 512)], hs.at[pl.ds(0, 512)])
        pltpu.sync_copy(stepc.at[pl.ds(go, 512)], hs.at[pl.ds(512, 512)])
        pltpu.sync_copy(done32.at[pl.ds(go, 512)], hs.at[pl.ds(1024, 512)])

        def ubody(v, carry):
            o = 16 * v
            a = hs[pl.ds(o, 16)]
            s = hs[pl.ds(512 + o, 16)]
            d = hs[pl.ds(1024 + o, 16)]
            stop = a == -1
            inc = jnp.where((d == 0) & (a != -1), c1, c0)
            ns = s + inc
            stp[pl.ds(o, 16)] = ns
            nd = jnp.where((d != 0) | stop | (ns >= MAX_STEPS), c1, c0)
            dne[pl.ds(o, 16)] = nd
            return carry

        _fori(512 // 16, ubody)
        pltpu.sync_copy(stp, step_o.at[pl.ds(go, 512)])
        pltpu.sync_copy(dne, done_o.at[pl.ds(go, 512)])

    if True:
        @pl.when(wid == 0)
        def _scatter():
            for cc in range(4):
                pltpu.sync_copy(curr.at[pl.ds(cc * 4096, 4096)],
                                tbl.at[pl.ds(cc * 4096, 4096)])
            for q in range(NQ):
                qo = q * QG
                pltpu.sync_copy(act.at[pl.ds(qo, QG)], hs.at[pl.ds(0, QG)])

                def ibody(v, carry):
                    a = hs[pl.ds(16 * v, 16)]
                    safe = jnp.where(a >= 0, a, 0)
                    idxq[pl.ds(16 * v, 16)] = safe
                    idxt[pl.ds(16 * v, 16)] = safe + E
                    return carry

                _fori(QG // 16, ibody)
                pltpu.async_copy(
                    eif.at[idxt], hs.at[pl.ds(QG, QG)], sem).wait()
                pltpu.async_copy(
                    eb.at[idxq], hs.at[pl.ds(2 * QG, QG)], sem).wait()

                def sbody(v, carry):
                    o = 16 * v
                    a = hs[pl.ds(o, 16)]
                    t_lo = hs[pl.ds(2 * QG + o, 16)]
                    t = jnp.where(a >= 0, t_lo, G)
                    comp = t * 16 + ii
                    sc_, lane = plsc.sort_key_val(comp, ii)
                    ts = sc_ >> 4
                    tmp16[...] = ts
                    tnxt = plsc.load_gather(tmp16, [jnp.minimum(ii + 1, 15)])
                    is_last = (ii == 15) | (ts != tnxt)
                    vals = plsc.load_gather(hs, [QG + o + lane])
                    plsc.store_scatter(tbl, [ts], vals, mask=is_last)
                    return carry

                _fori(QG // 16, sbody)
            pltpu.sync_copy(tbl.at[pl.ds(0, G)], curr_o)

    if True:
        w1 = wid - 1
        nc_mine = jnp.where(wid > 0, (NCHUNKS - w1 + NW - 2) // (NW - 1), 0)

        def in_descs(i):
            c = w1 + (NW - 1) * i
            po = (i % 2) * MCHUNK
            dh = pltpu.make_async_copy(
                eif.at[pl.ds(c * MCHUNK, MCHUNK)],
                hs.at[pl.ds(po, MCHUNK)], s_h)
            de = pltpu.make_async_copy(
                eb.at[pl.ds(c * MCHUNK, MCHUNK)],
                es.at[pl.ds(po, MCHUNK)], s_e)
            return dh, de

        def out_desc_p(i, p):
            c = w1 + (NW - 1) * i
            sem = [s_o0, s_o1][p]
            return pltpu.make_async_copy(
                ms.at[pl.ds(10 * p, 10)].bitcast(jnp.int8),
                mask_o.at[pl.ds((MCHUNK // 128) * c, MCHUNK // 128)],
                sem)

        @pl.when(wid > 0)
        def _prime():
            dh, de = in_descs(jnp.asarray(0, jnp.int32))
            dh.start()
            de.start()

        def mbody(i, carry):
            p = i % 2
            po = p * MCHUNK
            dh, de = in_descs(i)
            dh.wait()
            de.wait()

            @pl.when(i + 1 < nc_mine)
            def _next():
                nh, ne = in_descs(i + 1)
                nh.start()
                ne.start()

            @pl.when((i >= 2) & (p == 0))
            def _d0():
                out_desc_p(i - 2, 0).wait()

            @pl.when((i >= 2) & (p == 1))
            def _d1():
                out_desc_p(i - 2, 1).wait()

            def gbody(rr, carry2):
                for k in range(8):
                    cols = 16 * k + ii
                    word = jnp.zeros((L,), jnp.int32)
                    for b in range(4):
                        rows = po + 512 * rr + 128 * b + cols
                        bv = plsc.load_gather(es, [rows])
                        hv = plsc.load_gather(hs, [rows])
                        pv = plsc.load_gather(packed, [bv])
                        word = word | jnp.where(
                            hv == pv, jnp.asarray(1 << (8 * b), jnp.int32),
                            c0)
                    plsc.store_scatter(ms, [10 * p + rr + zc, cols], word)
                return carry2

            _fori(MCHUNK // 512, gbody)

            @pl.when(p == 0)
            def _s0():
                out_desc_p(i, 0).start()

            @pl.when(p == 1)
            def _s1():
                out_desc_p(i, 1).start()

            return carry

        _fori(nc_mine, mbody)

        @pl.when(wid > 0)
        def _final_drain():
            last = nc_mine - 1
            lp = last % 2

            @pl.when(lp == 0)
            def _f0():
                out_desc_p(last - 1, 1).wait()
                out_desc_p(last, 0).wait()

            @pl.when(lp == 1)
            def _f1():
                out_desc_p(last - 1, 0).wait()
                out_desc_p(last, 1).wait()


@jax.jit
def _run(edge_index, edge_batch, curr_nodes, step_counts, done, actions):
    eif = edge_index.reshape(2 * E).astype(jnp.int32)
    eb = edge_batch.astype(jnp.int32)
    curr = curr_nodes.astype(jnp.int32)
    stepc = step_counts.astype(jnp.int32)
    done32 = done.astype(jnp.int32)
    act = actions.astype(jnp.int32)

    mesh = plsc.VectorSubcoreMesh(core_axis_name="c", subcore_axis_name="s")
    out = pl.kernel(
        _body,
        out_type=[
            jax.ShapeDtypeStruct((E // 128, 128), jnp.int8),
            jax.ShapeDtypeStruct((G,), jnp.int32),
            jax.ShapeDtypeStruct((G,), jnp.int32),
            jax.ShapeDtypeStruct((G,), jnp.int32),
        ],
        mesh=mesh,
        compiler_params=pltpu.CompilerParams(needs_layout_passes=False),
        scratch_types=[
            pltpu.VMEM((G,), jnp.int32),
            pltpu.VMEM((6144,), jnp.int32),
            pltpu.VMEM((MCHUNK,), jnp.int32),
            pltpu.VMEM((2 * (MCHUNK // 512), 128), jnp.int32),
            pltpu.VMEM((G + 8,), jnp.int32),
            pltpu.VMEM((QG,), jnp.int32),
            pltpu.VMEM((QG,), jnp.int32),
            pltpu.VMEM((512,), jnp.int32),
            pltpu.VMEM((512,), jnp.int32),
            pltpu.VMEM((16,), jnp.int32),
            pltpu.SemaphoreType.DMA,
            pltpu.SemaphoreType.DMA,
            pltpu.SemaphoreType.DMA,
            pltpu.SemaphoreType.DMA,
            pltpu.SemaphoreType.DMA,
        ],
    )(eif, eb, curr, stepc, done32, act)
    edge_mask, new_curr, new_step, new_done = out
    return (edge_mask.reshape(E).astype(jnp.bool_),
            new_curr.astype(jnp.int64),
            new_step.astype(jnp.int64),
            new_done.astype(jnp.bool_))


def kernel(edge_index, edge_batch, node_ptr, curr_nodes, step_counts, done,
           actions):
    del node_ptr
    return _run(edge_index, edge_batch, curr_nodes, step_counts, done,
                actions)


# speedup vs baseline: 1.0114x; 1.0114x over previous
import jax
import jax.numpy as jnp
from jax import lax
from jax.experimental import pallas as pl
from jax.experimental.pallas import tpu as pltpu
from jax.experimental.pallas import tpu_sc as plsc

N_NODES = 100000
E = 3200000
G = 16384
MAX_STEPS = 10

NC = 2
NS = 16
NW = NC * NS
L = 16

MCHUNK = 5120
NCHUNKS = E // MCHUNK
QG = 2048
NQ = G // QG

def _fori(n, body):
    lax.fori_loop(jnp.asarray(0, jnp.int32), jnp.asarray(n, jnp.int32),
                  body, None)


def _body(eif, eb, curr, stepc, done32, act,
          mask_o, curr_o, step_o, done_o,
          packed, hs, es, ms, tbl, idxq, idxt, stp, dne, tmp16, sem):
    ii = lax.iota(jnp.int32, L)
    zc = jnp.zeros((L,), jnp.int32)
    c0 = jnp.asarray(0, jnp.int32)
    c1 = jnp.asarray(1, jnp.int32)
    wid = (jnp.asarray(lax.axis_index("s"), jnp.int32) * NC
           + jnp.asarray(lax.axis_index("c"), jnp.int32))

    if True:
        for pc in range(8):
            base = pc * 2048
            pltpu.sync_copy(curr.at[pl.ds(base, 2048)], hs.at[pl.ds(0, 2048)])
            pltpu.sync_copy(stepc.at[pl.ds(base, 2048)],
                            hs.at[pl.ds(2048, 2048)])
            pltpu.sync_copy(done32.at[pl.ds(base, 2048)],
                            hs.at[pl.ds(4096, 2048)])

            def pbody(v, carry, base=base):
                o = 16 * v
                c = hs[pl.ds(o, 16)]
                s = hs[pl.ds(2048 + o, 16)]
                d = hs[pl.ds(4096 + o, 16)]
                active = (d == 0) & (s < MAX_STEPS)
                packed[pl.ds(base + o, 16)] = jnp.where(active, c, -1)
                return carry

            _fori(2048 // 16, pbody)

    if True:
        go = wid * 512
        pltpu.sync_copy(act.at[pl.ds(go, 512)], hs.at[pl.ds(0, 512)])
        pltpu.sync_copy(stepc.at[pl.ds(go, 512)], hs.at[pl.ds(512, 512)])
        pltpu.sync_copy(done32.at[pl.ds(go, 512)], hs.at[pl.ds(1024, 512)])

        def ubody(v, carry):
            o = 16 * v
            a = hs[pl.ds(o, 16)]
            s = hs[pl.ds(512 + o, 16)]
            d = hs[pl.ds(1024 + o, 16)]
            stop = a == -1
            inc = jnp.where((d == 0) & (a != -1), c1, c0)
            ns = s + inc
            stp[pl.ds(o, 16)] = ns
            nd = jnp.where((d != 0) | stop | (ns >= MAX_STEPS), c1, c0)
            dne[pl.ds(o, 16)] = nd
            return carry

        _fori(512 // 16, ubody)
        pltpu.sync_copy(stp, step_o.at[pl.ds(go, 512)])
        pltpu.sync_copy(dne, done_o.at[pl.ds(go, 512)])

    if True:
        @pl.when(wid == 0)
        def _scatter():
            for cc in range(4):
                pltpu.sync_copy(curr.at[pl.ds(cc * 4096, 4096)],
                                tbl.at[pl.ds(cc * 4096, 4096)])
            for q in range(NQ):
                qo = q * QG
                pltpu.sync_copy(act.at[pl.ds(qo, QG)], hs.at[pl.ds(0, QG)])

                def ibody(v, carry):
                    a = hs[pl.ds(16 * v, 16)]
                    safe = jnp.where(a >= 0, a, 0)
                    idxq[pl.ds(16 * v, 16)] = safe
                    idxt[pl.ds(16 * v, 16)] = safe + E
                    return carry

                _fori(QG // 16, ibody)
                d1 = pltpu.async_copy(
                    eif.at[idxt], hs.at[pl.ds(QG, QG)], sem)
                d2 = pltpu.async_copy(
                    eb.at[idxq], hs.at[pl.ds(2 * QG, QG)], sem)
                d1.wait()
                d2.wait()

                def sbody(v, carry):
                    o = 16 * v
                    a = hs[pl.ds(o, 16)]
                    t_lo = hs[pl.ds(2 * QG + o, 16)]
                    t = jnp.where(a >= 0, t_lo, G)
                    comp = t * 16 + ii
                    sc_, lane = plsc.sort_key_val(comp, ii)
                    ts = sc_ >> 4
                    tmp16[...] = ts
                    tnxt = plsc.load_gather(tmp16, [jnp.minimum(ii + 1, 15)])
                    is_last = (ii == 15) | (ts != tnxt)
                    vals = plsc.load_gather(hs, [QG + o + lane])
                    plsc.store_scatter(tbl, [ts], vals, mask=is_last)
                    return carry

                _fori(QG // 16, sbody)
            pltpu.sync_copy(tbl.at[pl.ds(0, G)], curr_o)

    if True:
        w1 = wid - 1
        nc_mine = jnp.where(wid > 0, (NCHUNKS - w1 + NW - 2) // (NW - 1), 0)

        def mbody(i, carry):
            c = w1 + (NW - 1) * i
            start = c * MCHUNK
            pltpu.sync_copy(eif.at[pl.ds(start, MCHUNK)],
                            hs.at[pl.ds(0, MCHUNK)])
            pltpu.sync_copy(eb.at[pl.ds(start, MCHUNK)], es)


            def gbody(rr, carry2):
                for k in range(8):
                    cols = 16 * k + ii
                    word = jnp.zeros((L,), jnp.int32)
                    for b in range(4):
                        rows = 512 * rr + 128 * b + cols
                        bv = plsc.load_gather(es, [rows])
                        hv = plsc.load_gather(hs, [rows])
                        pv = plsc.load_gather(packed, [bv])
                        word = word | jnp.where(
                            hv == pv, jnp.asarray(1 << (8 * b), jnp.int32),
                            c0)
                    plsc.store_scatter(ms, [rr + zc, cols], word)
                return carry2

            _fori(MCHUNK // 512, gbody)
            pltpu.sync_copy(ms.bitcast(jnp.int8),
                            mask_o.at[pl.ds((MCHUNK // 128) * c,
                                            MCHUNK // 128)])
            return carry

        _fori(nc_mine, mbody)


@jax.jit
def _run(edge_index, edge_batch, curr_nodes, step_counts, done, actions):
    eif = edge_index.reshape(2 * E).astype(jnp.int32)
    eb = edge_batch.astype(jnp.int32)
    curr = curr_nodes.astype(jnp.int32)
    stepc = step_counts.astype(jnp.int32)
    done32 = done.astype(jnp.int32)
    act = actions.astype(jnp.int32)

    mesh = plsc.VectorSubcoreMesh(core_axis_name="c", subcore_axis_name="s")
    out = pl.kernel(
        _body,
        out_type=[
            jax.ShapeDtypeStruct((E // 128, 128), jnp.int8),
            jax.ShapeDtypeStruct((G,), jnp.int32),
            jax.ShapeDtypeStruct((G,), jnp.int32),
            jax.ShapeDtypeStruct((G,), jnp.int32),
        ],
        mesh=mesh,
        compiler_params=pltpu.CompilerParams(needs_layout_passes=False),
        scratch_types=[
            pltpu.VMEM((G,), jnp.int32),
            pltpu.VMEM((6144,), jnp.int32),
            pltpu.VMEM((MCHUNK,), jnp.int32),
            pltpu.VMEM((MCHUNK // 512, 128), jnp.int32),
            pltpu.VMEM((G + 8,), jnp.int32),
            pltpu.VMEM((QG,), jnp.int32),
            pltpu.VMEM((QG,), jnp.int32),
            pltpu.VMEM((512,), jnp.int32),
            pltpu.VMEM((512,), jnp.int32),
            pltpu.VMEM((16,), jnp.int32),
            pltpu.SemaphoreType.DMA,
        ],
    )(eif, eb, curr, stepc, done32, act)
    edge_mask, new_curr, new_step, new_done = out
    return (edge_mask.reshape(E).astype(jnp.bool_),
            new_curr.astype(jnp.int64),
            new_step.astype(jnp.int64),
            new_done.astype(jnp.bool_))


def kernel(edge_index, edge_batch, node_ptr, curr_nodes, step_counts, done,
           actions):
    del node_ptr
    return _run(edge_index, edge_batch, curr_nodes, step_counts, done,
                actions)
